# baseline (speedup 1.0000x reference)
import jax
import jax.numpy as jnp
from jax import lax
from jax.experimental import pallas as pl
from jax.experimental.pallas import tpu as pltpu

_N = 256
_B = 8
_C = 32


def _close_panel(p):
    for t in range(_B):
        p = jnp.minimum(p, p[:, t : t + 1] + p[t : t + 1, :])
    return p


def _fw_body(a_ref, o_ref, s_ref):
    n = _N
    nmat = a_ref.shape[0]
    rows = lax.broadcasted_iota(jnp.int32, (n, n), 0)
    cols = lax.broadcasted_iota(jnp.int32, (n, n), 1)
    eye = rows == cols
    for m in range(nmat):
        a = a_ref[m]
        w = jnp.where((a != 0.0) | eye, a, jnp.inf)
        o_ref[m] = jnp.where(eye, 0.0, w)

    def do_block(kb, load, store):
        base = kb * _B
        rs = []
        for m in range(nmat):
            p = pltpu.roll(load(m, pl.ds(base, _B)), -base, axis=1)
            rs.append(pltpu.roll(_close_panel(p), base, axis=1))
        for m in range(nmat):
            r = rs[m]
            for s in range(n // _C):
                d = load(m, pl.ds(s * _C, _C))
                c0 = pltpu.roll(d, -base, axis=1)[:, 0:_B]
                m0 = c0[:, 0:1] + r[0:1, :]
                m1 = c0[:, 1:2] + r[1:2, :]
                for t in range(2, _B, 2):
                    m0 = jnp.minimum(m0, c0[:, t : t + 1] + r[t : t + 1, :])
                    m1 = jnp.minimum(m1, c0[:, t + 1 : t + 2] + r[t + 1 : t + 2, :])
                store(m, pl.ds(s * _C, _C), jnp.minimum(d, jnp.minimum(m0, m1)))

    def load_o(m, ix):
        return o_ref[m, ix, :]

    def store_o(m, ix, v):
        o_ref[m, ix, :] = v

    def load_s(m, ix):
        return s_ref[m, ix, :]

    def store_s(m, ix, v):
        s_ref[m, ix, :] = v

    def block_pair(i, _):
        do_block(2 * i, load_o, store_s)
        do_block(2 * i + 1, load_s, store_o)
        return 0

    lax.fori_loop(0, n // (2 * _B), block_pair, 0)


def kernel(adj):
    n = adj.shape[-1]
    batch = adj.shape[0] * adj.shape[1]
    a = adj.reshape(batch, n, n)
    out = pl.pallas_call(
        _fw_body,
        out_shape=jax.ShapeDtypeStruct((batch, n, n), adj.dtype),
        scratch_shapes=[pltpu.VMEM((batch, n, n), jnp.float32)],
    )(a)
    return out.reshape(adj.shape)


# speedup vs baseline: 1.5667x; 1.5667x over previous
import jax
import jax.numpy as jnp
from jax import lax
from jax.experimental import pallas as pl
from jax.experimental.pallas import tpu as pltpu

_N = 256
_B = 8
_C = 32


def _close_panel(p):
    for t in range(_B):
        p = jnp.minimum(p, p[:, t : t + 1] + p[t : t + 1, :])
    return p


def _fw_body(a_ref, o_ref, s_ref):
    n = _N
    nmat = a_ref.shape[0]
    rows = lax.broadcasted_iota(jnp.int32, (n, n), 0)
    cols = lax.broadcasted_iota(jnp.int32, (n, n), 1)
    eye = rows == cols
    big = jnp.float32(1e18)
    for m in range(nmat):
        a = a_ref[m]
        w = jnp.where((a != 0.0) | eye, a, big)
        o_ref[m] = jnp.where(eye, 0.0, w)
    ones_row = jnp.ones((1, n), jnp.float32)

    def do_block(kb, load, store):
        base = kb * _B
        rs = []
        for m in range(nmat):
            p = pltpu.roll(load(m, pl.ds(base, _B)), -base, axis=1)
            rs.append(pltpu.roll(_close_panel(p), base, axis=1))
        for m in range(nmat):
            r = rs[m]
            for s in range(n // _C):
                d = load(m, pl.ds(s * _C, _C))
                c0 = pltpu.roll(d, -base, axis=1)[:, 0:_B]

                def outer(t):
                    colb = lax.dot_general(
                        c0[:, t : t + 1], ones_row,
                        (((1,), (0,)), ((), ())),
                        preferred_element_type=jnp.float32,
                    )
                    return colb + r[t : t + 1, :]

                m0 = outer(0)
                m1 = outer(1)
                for t in range(2, _B, 2):
                    m0 = jnp.minimum(m0, outer(t))
                    m1 = jnp.minimum(m1, outer(t + 1))
                store(m, pl.ds(s * _C, _C), jnp.minimum(d, jnp.minimum(m0, m1)))

    def load_o(m, ix):
        return o_ref[m, ix, :]

    def store_o(m, ix, v):
        o_ref[m, ix, :] = v

    def load_s(m, ix):
        return s_ref[m, ix, :]

    def store_s(m, ix, v):
        s_ref[m, ix, :] = v

    def block_pair(i, _):
        do_block(2 * i, load_o, store_s)
        do_block(2 * i + 1, load_s, store_o)
        return 0

    lax.fori_loop(0, n // (2 * _B), block_pair, 0)


def kernel(adj):
    n = adj.shape[-1]
    batch = adj.shape[0] * adj.shape[1]
    a = adj.reshape(batch, n, n)
    out = pl.pallas_call(
        _fw_body,
        out_shape=jax.ShapeDtypeStruct((batch, n, n), adj.dtype),
        scratch_shapes=[pltpu.VMEM((batch, n, n), jnp.float32)],
    )(a)
    return out.reshape(adj.shape)


# speedup vs baseline: 3.7280x; 2.3795x over previous
import jax
import jax.numpy as jnp
from jax import lax
from jax.experimental import pallas as pl
from jax.experimental.pallas import tpu as pltpu

_N = 256
_B = 32
_C = 64


def _close_panel(p):
    for t in range(_B):
        p = jnp.minimum(p, p[:, t : t + 1] + p[t : t + 1, :])
    return p


def _fw_body(a_ref, o_ref, s_ref):
    n = _N
    nmat = a_ref.shape[0]
    rows = lax.broadcasted_iota(jnp.int32, (n, n), 0)
    cols = lax.broadcasted_iota(jnp.int32, (n, n), 1)
    eye = rows == cols
    big = jnp.float32(1e18)
    for m in range(nmat):
        a = a_ref[m]
        w = jnp.where((a != 0.0) | eye, a, big)
        o_ref[m] = jnp.where(eye, 0.0, w)
    sel_iota = lax.broadcasted_iota(jnp.int32, (_B, n), 0)
    onehots = [
        jnp.where(sel_iota == t, jnp.float32(1.0), jnp.float32(0.0))
        for t in range(_B)
    ]

    def do_block(kb, load, loadl, store):
        base = kb * _B
        aligned = (kb // (128 // _B)) * 128
        off = base - aligned
        rs = []
        for m in range(nmat):
            p = pltpu.roll(load(m, pl.ds(base, _B)), -base, axis=1)
            rs.append(pltpu.roll(_close_panel(p), base, axis=1))
        for m in range(nmat):
            r = rs[m]
            for s in range(n // _C):
                d = load(m, pl.ds(s * _C, _C))
                dwin = loadl(m, pl.ds(s * _C, _C), pl.ds(aligned, 128))
                c0 = pltpu.roll(dwin, -off, axis=1)[:, 0:_B]

                def outer(t):
                    colb = lax.dot_general(
                        c0, onehots[t],
                        (((1,), (0,)), ((), ())),
                        preferred_element_type=jnp.float32,
                    )
                    return colb + r[t : t + 1, :]

                m0 = jnp.minimum(outer(0), outer(2))
                m1 = jnp.minimum(outer(1), outer(3))
                for t in range(4, _B, 2):
                    m0 = jnp.minimum(m0, outer(t))
                    m1 = jnp.minimum(m1, outer(t + 1))
                store(m, pl.ds(s * _C, _C),
                      jnp.minimum(d, jnp.minimum(m0, m1)))

    def load_o(m, ix):
        return o_ref[m, ix, :]

    def loadl_o(m, ix, lx):
        return o_ref[m, ix, lx]

    def store_o(m, ix, v):
        o_ref[m, ix, :] = v

    def load_s(m, ix):
        return s_ref[m, ix, :]

    def loadl_s(m, ix, lx):
        return s_ref[m, ix, lx]

    def store_s(m, ix, v):
        s_ref[m, ix, :] = v

    def block_pair(i, _):
        do_block(2 * i, load_o, loadl_o, store_s)
        do_block(2 * i + 1, load_s, loadl_s, store_o)
        return 0

    lax.fori_loop(0, n // (2 * _B), block_pair, 0)


def kernel(adj):
    n = adj.shape[-1]
    batch = adj.shape[0] * adj.shape[1]
    a = adj.reshape(batch, n, n)
    out = pl.pallas_call(
        _fw_body,
        out_shape=jax.ShapeDtypeStruct((batch, n, n), adj.dtype),
        scratch_shapes=[pltpu.VMEM((batch, n, n), jnp.float32)],
    )(a)
    return out.reshape(adj.shape)
